# baseline (speedup 1.0000x reference)
import functools

import jax
import jax.numpy as jnp
from jax import lax
from jax.experimental import pallas as pl
from jax.experimental.pallas import tpu as pltpu
from jax.experimental.pallas import tpu_sc as plsc

D = 64
B = 16384 * 50
NC, NS = 2, 16
NW = NC * NS
BPW = B // NW
CH = 1024
IPG = 128
NG = CH // IPG
NCHUNK = BPW // CH


@functools.lru_cache(maxsize=1)
def _build():
    mesh = plsc.VectorSubcoreMesh(core_axis_name="c", subcore_axis_name="s")

    @functools.partial(
        pl.kernel,
        mesh=mesh,
        out_type=jax.ShapeDtypeStruct((B, D), jnp.float32),
        compiler_params=pltpu.CompilerParams(use_tc_tiling_on_sc=False),
        scratch_types=[
            pltpu.VMEM((BPW,), jnp.int32),
            pltpu.VMEM((CH, D), jnp.float32),
            pltpu.SemaphoreType.DMA,
        ],
    )
    def gather_kernel(mat_hbm, idx_hbm, out_hbm, idx_v, rows_v, sem):
        wid = lax.axis_index("s") * NC + lax.axis_index("c")
        wbase = wid * BPW
        pltpu.sync_copy(idx_hbm.at[pl.ds(wbase, BPW)], idx_v)

        def chunk(g, carry):
            cbase = g * CH
            copies = []
            for j in range(NG):
                copies.append(pltpu.async_copy(
                    mat_hbm.at[idx_v.at[pl.ds(cbase + j * IPG, IPG)]],
                    rows_v.at[pl.ds(j * IPG, IPG)],
                    sem))
            for c in copies:
                c.wait()
            pltpu.sync_copy(rows_v, out_hbm.at[pl.ds(wbase + cbase, CH)])
            return carry

        lax.fori_loop(0, NCHUNK, chunk, 0)

    return gather_kernel


def kernel(x, mat):
    idx = x.reshape(-1).astype(jnp.int32)
    return _build()(mat, idx)
